# baseline (speedup 1.0000x reference)
import functools

import jax
import jax.numpy as jnp
from jax import lax
from jax.experimental import pallas as pl
from jax.experimental.pallas import tpu as pltpu
from jax.experimental.pallas import tpu_sc as plsc

_BATCH = 4096
_N_FIELDS = 26
_CARD = 200
_OUT_COLS = _N_FIELDS * _CARD

_info = plsc.get_sparse_core_info()
_NC, _NS, _L = _info.num_cores, _info.num_subcores, _info.num_lanes
_NW = _NC * _NS
_ROWS_PER_W = _BATCH // _NW
_CHUNK_ROWS = 16
_CHUNKS = _ROWS_PER_W // _CHUNK_ROWS
_XW = _CHUNK_ROWS * _N_FIELDS


@functools.partial(
    pl.kernel,
    out_type=jax.ShapeDtypeStruct((_BATCH, _OUT_COLS), jnp.int32),
    mesh=plsc.VectorSubcoreMesh(core_axis_name="c", subcore_axis_name="s"),
    compiler_params=pltpu.CompilerParams(needs_layout_passes=False),
    scratch_types=[
        pltpu.VMEM((_XW,), jnp.int32),
        pltpu.VMEM((_CHUNK_ROWS, _OUT_COLS), jnp.int32),
    ],
)
def _onehot_sc(x_hbm, out_hbm, xv, buf):
    wid = lax.axis_index("s") * _NC + lax.axis_index("c")
    row0 = wid * _ROWS_PER_W

    ones = jnp.ones((_L,), jnp.int32)
    zeros = jnp.zeros((_L,), jnp.int32)
    iota = lax.iota(jnp.int32, _L)

    def _zrow(r):
        def body(v, carry):
            buf[r, pl.ds(v * _L, _L)] = zeros
            return carry
        lax.fori_loop(0, _OUT_COLS // _L, body, 0)
    for r in range(_CHUNK_ROWS):
        _zrow(r)

    for c in range(_CHUNKS):
        base = row0 + c * _CHUNK_ROWS
        pltpu.sync_copy(x_hbm.at[pl.ds(base * _N_FIELDS, _XW)], xv)
        for v in range(_XW // _L):
            p = v * _L + iota
            r = p // _N_FIELDS
            col = (p - r * _N_FIELDS) * _CARD + xv[pl.ds(v * _L, _L)]
            plsc.store_scatter(buf, [r, col], ones)
        pltpu.sync_copy(buf, out_hbm.at[pl.ds(base, _CHUNK_ROWS)])
        for v in range(_XW // _L):
            p = v * _L + iota
            r = p // _N_FIELDS
            col = (p - r * _N_FIELDS) * _CARD + xv[pl.ds(v * _L, _L)]
            plsc.store_scatter(buf, [r, col], zeros)


def kernel(x):
    return _onehot_sc(x.reshape(-1))


# speedup vs baseline: 1.0856x; 1.0856x over previous
import functools

import jax
import jax.numpy as jnp
from jax import lax
from jax.experimental import pallas as pl
from jax.experimental.pallas import tpu as pltpu
from jax.experimental.pallas import tpu_sc as plsc

_BATCH = 4096
_N_FIELDS = 26
_CARD = 200
_OUT_COLS = _N_FIELDS * _CARD

_info = plsc.get_sparse_core_info()
_NC, _NS, _L = _info.num_cores, _info.num_subcores, _info.num_lanes
_NW = _NC * _NS
_ROWS_PER_W = _BATCH // _NW
_CHUNK_ROWS = 8
_CHUNKS = _ROWS_PER_W // _CHUNK_ROWS
_XW = _CHUNK_ROWS * _N_FIELDS
_XV = _XW // _L


@functools.partial(
    pl.kernel,
    out_type=jax.ShapeDtypeStruct((_BATCH, _OUT_COLS), jnp.int32),
    mesh=plsc.VectorSubcoreMesh(core_axis_name="c", subcore_axis_name="s"),
    compiler_params=pltpu.CompilerParams(needs_layout_passes=False),
    scratch_types=[
        pltpu.VMEM((_ROWS_PER_W * _N_FIELDS,), jnp.int32),
        pltpu.VMEM((_CHUNK_ROWS, _OUT_COLS), jnp.int32),
        pltpu.VMEM((_CHUNK_ROWS, _OUT_COLS), jnp.int32),
        pltpu.SemaphoreType.DMA,
        pltpu.SemaphoreType.DMA,
    ],
)
def _onehot_sc(x_hbm, z_hbm, out_hbm, xv, buf0, buf1, sem0, sem1):
    wid = lax.axis_index("s") * _NC + lax.axis_index("c")
    row0 = wid * _ROWS_PER_W
    bufs = (buf0, buf1)
    sems = (sem0, sem1)

    ones = jnp.ones((_L,), jnp.int32)
    zeros = jnp.zeros((_L,), jnp.int32)
    iota = lax.iota(jnp.int32, _L)

    xl = pltpu.async_copy(
        x_hbm.at[pl.ds(row0 * _N_FIELDS, _ROWS_PER_W * _N_FIELDS)], xv, sem0)
    pltpu.sync_copy(z_hbm, buf1)
    xl.wait()
    pltpu.sync_copy(z_hbm, buf0)

    rowv, colv = [], []
    for v in range(_XV):
        p = v * _L + iota
        r = p // _N_FIELDS
        rowv.append(r)
        colv.append((p - r * _N_FIELDS) * _CARD)

    dmas = [None, None]
    for c in range(_CHUNKS):
        s = c % 2
        if dmas[s] is not None:
            dmas[s].wait()
            for v in range(_XV):
                xs = xv[pl.ds((c - 2) * _XW + v * _L, _L)]
                plsc.store_scatter(bufs[s], [rowv[v], colv[v] + xs], zeros)
        for v in range(_XV):
            xs = xv[pl.ds(c * _XW + v * _L, _L)]
            plsc.store_scatter(bufs[s], [rowv[v], colv[v] + xs], ones)
        dmas[s] = pltpu.async_copy(
            bufs[s], out_hbm.at[pl.ds(row0 + c * _CHUNK_ROWS, _CHUNK_ROWS)],
            sems[s])
    dmas[0].wait()
    dmas[1].wait()


def kernel(x):
    z = jnp.zeros((_CHUNK_ROWS, _OUT_COLS), jnp.int32)
    return _onehot_sc(x.reshape(-1), z)
